# baseline (speedup 1.0000x reference)
import jax
import jax.numpy as jnp
from jax.experimental import pallas as pl
from jax.experimental.pallas import tpu as pltpu

BETA = 0.1
NQ = 4
K = 32


def _mm16(a16, b16):
    return jax.lax.dot_general(
        a16, b16, (((1,), (0,)), ((), ())), preferred_element_type=jnp.float32,
    )


def _mm(a, b16):
    return _mm16(a.astype(jnp.bfloat16), b16)


def _half(x, x16, w1, b1, w2, b2, dw1, db1, dw2, db2, cb1, cb2, cb3, cbst):
    h1 = jnp.maximum(_mm16(x16, w1) + b1, 0.0)
    res = _mm(h1, w2) + b2

    q_sum = jnp.zeros_like(res)
    q_err = jnp.zeros((1, 1), dtype=jnp.float32)
    ind_cols = []
    for i in range(NQ):
        cbt = cbst[i]
        cn = jnp.sum(cbt * cbt, axis=0, keepdims=True)
        rn = jnp.sum(res * res, axis=1, keepdims=True)
        dist = rn - 2.0 * _mm(res, cbt.astype(jnp.bfloat16)) + cn
        dmin = jnp.min(dist, axis=1, keepdims=True)
        iota = jax.lax.broadcasted_iota(jnp.int32, dist.shape, 1)
        masked = jnp.where(dist == dmin, iota, jnp.int32(K))
        ind = jnp.min(masked, axis=1, keepdims=True)
        one_hot = (iota == ind).astype(jnp.bfloat16)
        quant = (_mm16(one_hot, cb1[i]) + _mm16(one_hot, cb2[i])) \
            + _mm16(one_hot, cb3[i])
        q_sum = q_sum + quant
        res = res - quant
        q_err = q_err + jnp.sum(res * res, keepdims=True).reshape(1, 1)
        ind_cols.append(ind)

    h2 = jnp.maximum(_mm(q_sum, dw1) + db1, 0.0)
    xr = _mm(h2, dw2) + db2
    rec = jnp.sum((xr - x) ** 2, keepdims=True).reshape(1, 1)
    d_out = jnp.float32(q_sum.shape[1])
    part = rec + q_err * ((1.0 + BETA) / d_out)
    return jnp.concatenate(ind_cols, axis=1), part


def _rqvae_body(x_ref, x16_ref, w1_ref, b1_ref, w2_ref, b2_ref, dw1_ref,
                db1_ref, dw2_ref, db2_ref, cb1_ref, cb2_ref, cb3_ref,
                cbst_ref, loss_ref, inds_ref):
    ws = (w1_ref[...], b1_ref[...], w2_ref[...], b2_ref[...], dw1_ref[...],
          db1_ref[...], dw2_ref[...], db2_ref[...], cb1_ref, cb2_ref, cb3_ref,
          cbst_ref)
    bt = x_ref.shape[0]
    hb = bt // 2
    inds_a, part_a = _half(x_ref[:hb], x16_ref[:hb], *ws)
    inds_b, part_b = _half(x_ref[hb:], x16_ref[hb:], *ws)
    part = part_a + part_b
    inds_ref[...] = jnp.concatenate([inds_a, inds_b], axis=0)

    @pl.when(pl.program_id(0) == 0)
    def _init():
        loss_ref[...] = part

    @pl.when(pl.program_id(0) != 0)
    def _acc():
        loss_ref[...] = loss_ref[...] + part


@jax.jit
def kernel(x, enc_w1, enc_b1, enc_w2, enc_b2, dec_w1, dec_b1, dec_w2, dec_b2,
           codebooks):
    B, D_IN = x.shape
    H = enc_w1.shape[1]
    D_OUT = enc_w2.shape[1]
    BT = 1024
    grid = (B // BT,)

    cbs_t = jnp.swapaxes(codebooks, 1, 2)
    cb1 = jax.lax.optimization_barrier(codebooks.astype(jnp.bfloat16))
    r1 = codebooks - cb1.astype(jnp.float32)
    cb2 = jax.lax.optimization_barrier(r1.astype(jnp.bfloat16))
    cb3 = (r1 - cb2.astype(jnp.float32)).astype(jnp.bfloat16)

    const = lambda *_: (0, 0)
    const3 = lambda *_: (0, 0, 0)
    loss2d, inds_bt = pl.pallas_call(
        _rqvae_body,
        grid=grid,
        in_specs=[
            pl.BlockSpec((BT, D_IN), lambda i: (i, 0)),
            pl.BlockSpec((BT, D_IN), lambda i: (i, 0)),
            pl.BlockSpec((D_IN, H), const),
            pl.BlockSpec((1, H), const),
            pl.BlockSpec((H, D_OUT), const),
            pl.BlockSpec((1, D_OUT), const),
            pl.BlockSpec((D_OUT, H), const),
            pl.BlockSpec((1, H), const),
            pl.BlockSpec((H, D_IN), const),
            pl.BlockSpec((1, D_IN), const),
            pl.BlockSpec((NQ, K, D_OUT), const3),
            pl.BlockSpec((NQ, K, D_OUT), const3),
            pl.BlockSpec((NQ, K, D_OUT), const3),
            pl.BlockSpec((NQ, D_OUT, K), const3),
        ],
        out_specs=[
            pl.BlockSpec((1, 1), const),
            pl.BlockSpec((BT, NQ), lambda i: (i, 0)),
        ],
        out_shape=[
            jax.ShapeDtypeStruct((1, 1), jnp.float32),
            jax.ShapeDtypeStruct((B, NQ), jnp.int32),
        ],
        compiler_params=pltpu.CompilerParams(
            dimension_semantics=("arbitrary",),
            vmem_limit_bytes=60 * 1024 * 1024,
        ),
    )(
        x, x.astype(jnp.bfloat16),
        enc_w1.astype(jnp.bfloat16), enc_b1.reshape(1, H),
        enc_w2.astype(jnp.bfloat16), enc_b2.reshape(1, D_OUT),
        dec_w1.astype(jnp.bfloat16), dec_b1.reshape(1, H),
        dec_w2.astype(jnp.bfloat16), dec_b2.reshape(1, D_IN),
        cb1, cb2, cb3, cbs_t,
    )

    loss = loss2d[0, 0] / jnp.float32(B)
    return (loss, inds_bt.T)


# speedup vs baseline: 1.0988x; 1.0988x over previous
import jax
import jax.numpy as jnp
from jax.experimental import pallas as pl
from jax.experimental.pallas import tpu as pltpu

BETA = 0.1
NQ = 4
K = 32


def _mm16(a16, b16):
    return jax.lax.dot_general(
        a16, b16, (((1,), (0,)), ((), ())), preferred_element_type=jnp.float32,
    )


def _mm(a, b16):
    return _mm16(a.astype(jnp.bfloat16), b16)


def _half(x, x16, w1, b1, w2, b2, dw1, db1, dw2, db2, cb1, cb2, cb3, cbst):
    h1 = jnp.maximum(_mm16(x16, w1) + b1, 0.0)
    res = _mm(h1, w2) + b2

    q_sum = jnp.zeros_like(res)
    q_err = jnp.zeros((1, 1), dtype=jnp.float32)
    ind_cols = []
    for i in range(NQ):
        cbt = cbst[i]
        cn = jnp.sum(cbt * cbt, axis=0, keepdims=True)
        rn = jnp.sum(res * res, axis=1, keepdims=True)
        dist = rn - 2.0 * _mm(res, cbt.astype(jnp.bfloat16)) + cn
        dmin = jnp.min(dist, axis=1, keepdims=True)
        iota = jax.lax.broadcasted_iota(jnp.int32, dist.shape, 1)
        masked = jnp.where(dist == dmin, iota, jnp.int32(K))
        ind = jnp.min(masked, axis=1, keepdims=True)
        one_hot = (iota == ind).astype(jnp.bfloat16)
        quant = (_mm16(one_hot, cb1[i]) + _mm16(one_hot, cb2[i])) \
            + _mm16(one_hot, cb3[i])
        q_sum = q_sum + quant
        res = res - quant
        q_err = q_err + jnp.sum(res * res, keepdims=True).reshape(1, 1)
        ind_cols.append(ind)

    h2 = jnp.maximum(_mm(q_sum, dw1) + db1, 0.0)
    xr = _mm(h2, dw2) + db2
    rec = jnp.sum((xr - x) ** 2, keepdims=True).reshape(1, 1)
    d_out = jnp.float32(q_sum.shape[1])
    part = rec + q_err * ((1.0 + BETA) / d_out)
    return jnp.concatenate(ind_cols, axis=1), part


def _rqvae_body(x_ref, x16_ref, w1_ref, b1_ref, w2_ref, b2_ref, dw1_ref,
                db1_ref, dw2_ref, db2_ref, cb1_ref, cb2_ref, cb3_ref,
                cbst_ref, loss_ref, inds_ref):
    ws = (w1_ref[...], b1_ref[...], w2_ref[...], b2_ref[...], dw1_ref[...],
          db1_ref[...], dw2_ref[...], db2_ref[...], cb1_ref, cb2_ref, cb3_ref,
          cbst_ref)
    inds, part = _half(x_ref[...], x16_ref[...], *ws)
    inds_ref[...] = inds

    @pl.when(pl.program_id(0) == 0)
    def _init():
        loss_ref[...] = part

    @pl.when(pl.program_id(0) != 0)
    def _acc():
        loss_ref[...] = loss_ref[...] + part


@jax.jit
def kernel(x, enc_w1, enc_b1, enc_w2, enc_b2, dec_w1, dec_b1, dec_w2, dec_b2,
           codebooks):
    B, D_IN = x.shape
    H = enc_w1.shape[1]
    D_OUT = enc_w2.shape[1]
    BT = 1024
    grid = (B // BT,)

    cbs_t = jnp.swapaxes(codebooks, 1, 2)
    cb1 = jax.lax.optimization_barrier(codebooks.astype(jnp.bfloat16))
    r1 = codebooks - cb1.astype(jnp.float32)
    cb2 = jax.lax.optimization_barrier(r1.astype(jnp.bfloat16))
    cb3 = (r1 - cb2.astype(jnp.float32)).astype(jnp.bfloat16)

    const = lambda *_: (0, 0)
    const3 = lambda *_: (0, 0, 0)
    loss2d, inds_bt = pl.pallas_call(
        _rqvae_body,
        grid=grid,
        in_specs=[
            pl.BlockSpec((BT, D_IN), lambda i: (i, 0)),
            pl.BlockSpec((BT, D_IN), lambda i: (i, 0)),
            pl.BlockSpec((D_IN, H), const),
            pl.BlockSpec((1, H), const),
            pl.BlockSpec((H, D_OUT), const),
            pl.BlockSpec((1, D_OUT), const),
            pl.BlockSpec((D_OUT, H), const),
            pl.BlockSpec((1, H), const),
            pl.BlockSpec((H, D_IN), const),
            pl.BlockSpec((1, D_IN), const),
            pl.BlockSpec((NQ, K, D_OUT), const3),
            pl.BlockSpec((NQ, K, D_OUT), const3),
            pl.BlockSpec((NQ, K, D_OUT), const3),
            pl.BlockSpec((NQ, D_OUT, K), const3),
        ],
        out_specs=[
            pl.BlockSpec((1, 1), const),
            pl.BlockSpec((BT, NQ), lambda i: (i, 0)),
        ],
        out_shape=[
            jax.ShapeDtypeStruct((1, 1), jnp.float32),
            jax.ShapeDtypeStruct((B, NQ), jnp.int32),
        ],
        compiler_params=pltpu.CompilerParams(
            dimension_semantics=("arbitrary",),
            vmem_limit_bytes=60 * 1024 * 1024,
        ),
    )(
        x, x.astype(jnp.bfloat16),
        enc_w1.astype(jnp.bfloat16), enc_b1.reshape(1, H),
        enc_w2.astype(jnp.bfloat16), enc_b2.reshape(1, D_OUT),
        dec_w1.astype(jnp.bfloat16), dec_b1.reshape(1, H),
        dec_w2.astype(jnp.bfloat16), dec_b2.reshape(1, D_IN),
        cb1, cb2, cb3, cbs_t,
    )

    loss = loss2d[0, 0] / jnp.float32(B)
    return (loss, inds_bt.T)


# speedup vs baseline: 1.1931x; 1.0858x over previous
import jax
import jax.numpy as jnp
from jax.experimental import pallas as pl
from jax.experimental.pallas import tpu as pltpu

BETA = 0.1
NQ = 4
K = 32


def _mm16(a16, b16):
    return jax.lax.dot_general(
        a16, b16, (((1,), (0,)), ((), ())), preferred_element_type=jnp.float32,
    )


def _mm(a, b16):
    return _mm16(a.astype(jnp.bfloat16), b16)


def _half(x, x16, w1, b1, w2, b2, dw1, db1, dw2, db2, cb1, cb2, cb3, cbst):
    h1 = jnp.maximum(_mm16(x16, w1) + b1, 0.0)
    res = _mm(h1, w2) + b2

    q_sum = jnp.zeros_like(res)
    rn = jnp.sum(res * res, axis=1, keepdims=True)
    q_err_rows = jnp.zeros_like(rn)
    ind_cols = []
    for i in range(NQ):
        cbt = cbst[i]
        cn = jnp.sum(cbt * cbt, axis=0, keepdims=True)
        dist = rn - 2.0 * _mm(res, cbt.astype(jnp.bfloat16)) + cn
        dmin = jnp.min(dist, axis=1, keepdims=True)
        iota = jax.lax.broadcasted_iota(jnp.int32, dist.shape, 1)
        masked = jnp.where(dist == dmin, iota, jnp.int32(K))
        ind = jnp.min(masked, axis=1, keepdims=True)
        one_hot = (iota == ind).astype(jnp.bfloat16)
        quant = (_mm16(one_hot, cb1[i]) + _mm16(one_hot, cb2[i])) \
            + _mm16(one_hot, cb3[i])
        q_sum = q_sum + quant
        res = res - quant
        rn = jnp.sum(res * res, axis=1, keepdims=True)
        q_err_rows = q_err_rows + rn
        ind_cols.append(ind)

    h2 = jnp.maximum(_mm(q_sum, dw1) + db1, 0.0)
    xr = _mm(h2, dw2) + db2
    rec = jnp.sum((xr - x) ** 2, keepdims=True).reshape(1, 1)
    q_err = jnp.sum(q_err_rows, keepdims=True).reshape(1, 1)
    d_out = jnp.float32(q_sum.shape[1])
    part = rec + q_err * ((1.0 + BETA) / d_out)
    return jnp.concatenate(ind_cols, axis=1), part


def _rqvae_body(x_ref, w1_ref, b1_ref, w2_ref, b2_ref, dw1_ref,
                db1_ref, dw2_ref, db2_ref, cb1_ref, cb2_ref, cb3_ref,
                cbst_ref, loss_ref, inds_ref):
    ws = (w1_ref[...], b1_ref[...], w2_ref[...], b2_ref[...], dw1_ref[...],
          db1_ref[...], dw2_ref[...], db2_ref[...], cb1_ref, cb2_ref, cb3_ref,
          cbst_ref)
    x = x_ref[...]
    inds, part = _half(x, x.astype(jnp.bfloat16), *ws)
    inds_ref[...] = inds

    @pl.when(pl.program_id(0) == 0)
    def _init():
        loss_ref[...] = part

    @pl.when(pl.program_id(0) != 0)
    def _acc():
        loss_ref[...] = loss_ref[...] + part


@jax.jit
def kernel(x, enc_w1, enc_b1, enc_w2, enc_b2, dec_w1, dec_b1, dec_w2, dec_b2,
           codebooks):
    B, D_IN = x.shape
    H = enc_w1.shape[1]
    D_OUT = enc_w2.shape[1]
    BT = 1024
    grid = (B // BT,)

    cbs_t = jnp.swapaxes(codebooks, 1, 2)
    cb1 = jax.lax.optimization_barrier(codebooks.astype(jnp.bfloat16))
    r1 = codebooks - cb1.astype(jnp.float32)
    cb2 = jax.lax.optimization_barrier(r1.astype(jnp.bfloat16))
    cb3 = (r1 - cb2.astype(jnp.float32)).astype(jnp.bfloat16)

    const = lambda *_: (0, 0)
    const3 = lambda *_: (0, 0, 0)
    loss2d, inds_bt = pl.pallas_call(
        _rqvae_body,
        grid=grid,
        in_specs=[
            pl.BlockSpec((BT, D_IN), lambda i: (i, 0)),
            pl.BlockSpec((D_IN, H), const),
            pl.BlockSpec((1, H), const),
            pl.BlockSpec((H, D_OUT), const),
            pl.BlockSpec((1, D_OUT), const),
            pl.BlockSpec((D_OUT, H), const),
            pl.BlockSpec((1, H), const),
            pl.BlockSpec((H, D_IN), const),
            pl.BlockSpec((1, D_IN), const),
            pl.BlockSpec((NQ, K, D_OUT), const3),
            pl.BlockSpec((NQ, K, D_OUT), const3),
            pl.BlockSpec((NQ, K, D_OUT), const3),
            pl.BlockSpec((NQ, D_OUT, K), const3),
        ],
        out_specs=[
            pl.BlockSpec((1, 1), const),
            pl.BlockSpec((BT, NQ), lambda i: (i, 0)),
        ],
        out_shape=[
            jax.ShapeDtypeStruct((1, 1), jnp.float32),
            jax.ShapeDtypeStruct((B, NQ), jnp.int32),
        ],
        compiler_params=pltpu.CompilerParams(
            dimension_semantics=("arbitrary",),
            vmem_limit_bytes=60 * 1024 * 1024,
        ),
    )(
        x,
        enc_w1.astype(jnp.bfloat16), enc_b1.reshape(1, H),
        enc_w2.astype(jnp.bfloat16), enc_b2.reshape(1, D_OUT),
        dec_w1.astype(jnp.bfloat16), dec_b1.reshape(1, H),
        dec_w2.astype(jnp.bfloat16), dec_b2.reshape(1, D_IN),
        cb1, cb2, cb3, cbs_t,
    )

    loss = loss2d[0, 0] / jnp.float32(B)
    return (loss, inds_bt.T)


# speedup vs baseline: 1.2300x; 1.0309x over previous
import jax
import jax.numpy as jnp
from jax.experimental import pallas as pl
from jax.experimental.pallas import tpu as pltpu

BETA = 0.1
NQ = 4
K = 32


def _mm16(a16, b16):
    return jax.lax.dot_general(
        a16, b16, (((1,), (0,)), ((), ())), preferred_element_type=jnp.float32,
    )


def _mm(a, b16):
    return _mm16(a.astype(jnp.bfloat16), b16)


def _half(x, x16, w1, b1, w2, b2, dw1, db1, dw2, db2, cb1, cb2, cb3, cbst):
    h1 = jnp.maximum(_mm16(x16, w1) + b1, 0.0)
    res = _mm(h1, w2) + b2

    q_sum = jnp.zeros_like(res)
    rn = jnp.sum(res * res, axis=1, keepdims=True)
    q_err_rows = jnp.zeros_like(rn)
    iota = jax.lax.broadcasted_iota(
        jnp.int32, (x.shape[0], K), 1).astype(jnp.float32)
    ind_cols = []
    for i in range(NQ):
        cbt = cbst[i]
        cn = jnp.sum(cbt * cbt, axis=0, keepdims=True)
        dist = rn - 2.0 * _mm(res, cbt.astype(jnp.bfloat16)) + cn
        dmin = jnp.min(dist, axis=1, keepdims=True)
        masked = jnp.where(dist == dmin, iota, jnp.float32(K))
        ind = jnp.min(masked, axis=1, keepdims=True)
        one_hot = (iota == ind).astype(jnp.bfloat16)
        quant = (_mm16(one_hot, cb1[i]) + _mm16(one_hot, cb2[i])) \
            + _mm16(one_hot, cb3[i])
        q_sum = q_sum + quant
        res = res - quant
        rn = jnp.sum(res * res, axis=1, keepdims=True)
        q_err_rows = q_err_rows + rn
        ind_cols.append(ind)

    h2 = jnp.maximum(_mm(q_sum, dw1) + db1, 0.0)
    xr = _mm(h2, dw2) + db2
    rec = jnp.sum((xr - x) ** 2, keepdims=True).reshape(1, 1)
    q_err = jnp.sum(q_err_rows, keepdims=True).reshape(1, 1)
    d_out = jnp.float32(q_sum.shape[1])
    part = rec + q_err * ((1.0 + BETA) / d_out)
    return jnp.concatenate(ind_cols, axis=1).astype(jnp.int32), part


def _rqvae_body(x_ref, w1_ref, b1_ref, w2_ref, b2_ref, dw1_ref,
                db1_ref, dw2_ref, db2_ref, cb1_ref, cb2_ref, cb3_ref,
                cbst_ref, loss_ref, inds_ref):
    ws = (w1_ref[...], b1_ref[...], w2_ref[...], b2_ref[...], dw1_ref[...],
          db1_ref[...], dw2_ref[...], db2_ref[...], cb1_ref, cb2_ref, cb3_ref,
          cbst_ref)
    x = x_ref[...]
    inds, part = _half(x, x.astype(jnp.bfloat16), *ws)
    inds_ref[...] = inds

    @pl.when(pl.program_id(0) == 0)
    def _init():
        loss_ref[...] = part

    @pl.when(pl.program_id(0) != 0)
    def _acc():
        loss_ref[...] = loss_ref[...] + part


@jax.jit
def kernel(x, enc_w1, enc_b1, enc_w2, enc_b2, dec_w1, dec_b1, dec_w2, dec_b2,
           codebooks):
    B, D_IN = x.shape
    H = enc_w1.shape[1]
    D_OUT = enc_w2.shape[1]
    BT = 1024
    grid = (B // BT,)

    cbs_t = jnp.swapaxes(codebooks, 1, 2)
    cb1 = jax.lax.optimization_barrier(codebooks.astype(jnp.bfloat16))
    r1 = codebooks - cb1.astype(jnp.float32)
    cb2 = jax.lax.optimization_barrier(r1.astype(jnp.bfloat16))
    cb3 = (r1 - cb2.astype(jnp.float32)).astype(jnp.bfloat16)

    const = lambda *_: (0, 0)
    const3 = lambda *_: (0, 0, 0)
    loss2d, inds_bt = pl.pallas_call(
        _rqvae_body,
        grid=grid,
        in_specs=[
            pl.BlockSpec((BT, D_IN), lambda i: (i, 0)),
            pl.BlockSpec((D_IN, H), const),
            pl.BlockSpec((1, H), const),
            pl.BlockSpec((H, D_OUT), const),
            pl.BlockSpec((1, D_OUT), const),
            pl.BlockSpec((D_OUT, H), const),
            pl.BlockSpec((1, H), const),
            pl.BlockSpec((H, D_IN), const),
            pl.BlockSpec((1, D_IN), const),
            pl.BlockSpec((NQ, K, D_OUT), const3),
            pl.BlockSpec((NQ, K, D_OUT), const3),
            pl.BlockSpec((NQ, K, D_OUT), const3),
            pl.BlockSpec((NQ, D_OUT, K), const3),
        ],
        out_specs=[
            pl.BlockSpec((1, 1), const),
            pl.BlockSpec((BT, NQ), lambda i: (i, 0)),
        ],
        out_shape=[
            jax.ShapeDtypeStruct((1, 1), jnp.float32),
            jax.ShapeDtypeStruct((B, NQ), jnp.int32),
        ],
        compiler_params=pltpu.CompilerParams(
            dimension_semantics=("arbitrary",),
            vmem_limit_bytes=64 * 1024 * 1024,
        ),
    )(
        x,
        enc_w1.astype(jnp.bfloat16), enc_b1.reshape(1, H),
        enc_w2.astype(jnp.bfloat16), enc_b2.reshape(1, D_OUT),
        dec_w1.astype(jnp.bfloat16), dec_b1.reshape(1, H),
        dec_w2.astype(jnp.bfloat16), dec_b2.reshape(1, D_IN),
        cb1, cb2, cb3, cbs_t,
    )

    loss = loss2d[0, 0] / jnp.float32(B)
    return (loss, inds_bt.T)


# speedup vs baseline: 1.2760x; 1.0374x over previous
import jax
import jax.numpy as jnp
from jax.experimental import pallas as pl
from jax.experimental.pallas import tpu as pltpu

BETA = 0.1
NQ = 4
K = 32


def _mm16(a16, b16):
    return jax.lax.dot_general(
        a16, b16, (((1,), (0,)), ((), ())), preferred_element_type=jnp.float32,
    )


def _mm(a, b16):
    return _mm16(a.astype(jnp.bfloat16), b16)


def _half(x, x16, w1, b1, w2, b2, dw1, db1, dw2, db2, cb1, cb2, cb3, cbst):
    h1 = jnp.maximum(_mm16(x16, w1) + b1, 0.0)
    res = _mm(h1, w2) + b2

    q_sum = jnp.zeros_like(res)
    rn = jnp.sum(res * res, axis=1, keepdims=True)
    q_err_rows = jnp.zeros_like(rn)
    iota = jax.lax.broadcasted_iota(
        jnp.int32, (x.shape[0], K), 1).astype(jnp.float32)
    ind_cols = []
    for i in range(NQ):
        cbt = cbst[i]
        cn = jnp.sum(cbt * cbt, axis=0, keepdims=True)
        dist = rn - 2.0 * _mm(res, cbt.astype(jnp.bfloat16)) + cn
        dmin = jnp.min(dist, axis=1, keepdims=True)
        masked = jnp.where(dist == dmin, iota, jnp.float32(K))
        ind = jnp.min(masked, axis=1, keepdims=True)
        one_hot = (iota == ind).astype(jnp.bfloat16)
        quant = (_mm16(one_hot, cb1[i]) + _mm16(one_hot, cb2[i])) \
            + _mm16(one_hot, cb3[i])
        q_sum = q_sum + quant
        res = res - quant
        rn = jnp.sum(res * res, axis=1, keepdims=True)
        q_err_rows = q_err_rows + rn
        ind_cols.append(ind)

    h2 = jnp.maximum(_mm(q_sum, dw1) + db1, 0.0)
    xr = _mm(h2, dw2) + db2
    rec = jnp.sum((xr - x) ** 2, keepdims=True).reshape(1, 1)
    q_err = jnp.sum(q_err_rows, keepdims=True).reshape(1, 1)
    d_out = jnp.float32(q_sum.shape[1])
    part = rec + q_err * ((1.0 + BETA) / d_out)
    return jnp.concatenate(ind_cols, axis=1).astype(jnp.int32), part


def _rqvae_body(x_ref, w1_ref, b1_ref, w2_ref, b2_ref, dw1_ref,
                db1_ref, dw2_ref, db2_ref, cb1_ref, cb2_ref, cb3_ref,
                cbst_ref, loss_ref, inds_ref):
    ws = (w1_ref[...], b1_ref[...], w2_ref[...], b2_ref[...], dw1_ref[...],
          db1_ref[...], dw2_ref[...], db2_ref[...], cb1_ref, cb2_ref, cb3_ref,
          cbst_ref)
    x = x_ref[...]
    inds, part = _half(x, x.astype(jnp.bfloat16), *ws)
    inds_ref[...] = inds

    @pl.when(pl.program_id(0) == 0)
    def _init():
        loss_ref[...] = part

    @pl.when(pl.program_id(0) != 0)
    def _acc():
        loss_ref[...] = loss_ref[...] + part


@jax.jit
def kernel(x, enc_w1, enc_b1, enc_w2, enc_b2, dec_w1, dec_b1, dec_w2, dec_b2,
           codebooks):
    B, D_IN = x.shape
    H = enc_w1.shape[1]
    D_OUT = enc_w2.shape[1]
    BT = 2048
    grid = (B // BT,)

    cbs_t = jnp.swapaxes(codebooks, 1, 2)
    cb1 = jax.lax.optimization_barrier(codebooks.astype(jnp.bfloat16))
    r1 = codebooks - cb1.astype(jnp.float32)
    cb2 = jax.lax.optimization_barrier(r1.astype(jnp.bfloat16))
    cb3 = (r1 - cb2.astype(jnp.float32)).astype(jnp.bfloat16)

    const = lambda *_: (0, 0)
    const3 = lambda *_: (0, 0, 0)
    loss2d, inds_bt = pl.pallas_call(
        _rqvae_body,
        grid=grid,
        in_specs=[
            pl.BlockSpec((BT, D_IN), lambda i: (i, 0)),
            pl.BlockSpec((D_IN, H), const),
            pl.BlockSpec((1, H), const),
            pl.BlockSpec((H, D_OUT), const),
            pl.BlockSpec((1, D_OUT), const),
            pl.BlockSpec((D_OUT, H), const),
            pl.BlockSpec((1, H), const),
            pl.BlockSpec((H, D_IN), const),
            pl.BlockSpec((1, D_IN), const),
            pl.BlockSpec((NQ, K, D_OUT), const3),
            pl.BlockSpec((NQ, K, D_OUT), const3),
            pl.BlockSpec((NQ, K, D_OUT), const3),
            pl.BlockSpec((NQ, D_OUT, K), const3),
        ],
        out_specs=[
            pl.BlockSpec((1, 1), const),
            pl.BlockSpec((BT, NQ), lambda i: (i, 0)),
        ],
        out_shape=[
            jax.ShapeDtypeStruct((1, 1), jnp.float32),
            jax.ShapeDtypeStruct((B, NQ), jnp.int32),
        ],
        compiler_params=pltpu.CompilerParams(
            dimension_semantics=("arbitrary",),
            vmem_limit_bytes=64 * 1024 * 1024,
        ),
    )(
        x,
        enc_w1.astype(jnp.bfloat16), enc_b1.reshape(1, H),
        enc_w2.astype(jnp.bfloat16), enc_b2.reshape(1, D_OUT),
        dec_w1.astype(jnp.bfloat16), dec_b1.reshape(1, H),
        dec_w2.astype(jnp.bfloat16), dec_b2.reshape(1, D_IN),
        cb1, cb2, cb3, cbs_t,
    )

    loss = loss2d[0, 0] / jnp.float32(B)
    return (loss, inds_bt.T)
